# baseline (speedup 1.0000x reference)
import jax
import jax.numpy as jnp
from jax import lax
from jax.experimental import pallas as pl
from jax.experimental.pallas import tpu as pltpu

_D_MODEL = 4096
_NUM_EXPERTS = 64
_BLOCK_T = 1024


def _router_kernel(x_ref, w_ref, s_ref, g_ref):
    h = _BLOCK_T // 16
    w = w_ref[...]
    for p in range(16):
        sl = pl.ds(p * h, h)
        logits_all = lax.dot_general(
            x_ref[sl, :], w, (((1,), (1,)), ((), ())),
            preferred_element_type=jnp.float32)
        logits = logits_all[:, :_NUM_EXPERTS]
        m = jnp.max(logits, axis=-1, keepdims=True)
        e = jnp.exp(logits - m)
        s_ref[sl, :] = e / jnp.sum(e, axis=-1, keepdims=True)
        g_ref[sl, :] = jax.nn.sigmoid(logits_all[:, _NUM_EXPERTS:_NUM_EXPERTS + 1])


def kernel(x, W_router, W_shared_gate):
    tokens, d = x.shape
    n_exp = W_router.shape[0]
    w_all = jnp.concatenate(
        [W_router, W_shared_gate,
         jnp.zeros((128 - n_exp - 1, d), dtype=x.dtype)], axis=0)

    grid = (tokens // _BLOCK_T,)
    s, g = pl.pallas_call(
        _router_kernel,
        grid=grid,
        in_specs=[
            pl.BlockSpec((_BLOCK_T, d), lambda i: (i, 0)),
            pl.BlockSpec((128, d), lambda i: (0, 0)),
        ],
        out_specs=[
            pl.BlockSpec((_BLOCK_T, n_exp), lambda i: (i, 0)),
            pl.BlockSpec((_BLOCK_T, 1), lambda i: (i, 0)),
        ],
        out_shape=[
            jax.ShapeDtypeStruct((tokens, n_exp), x.dtype),
            jax.ShapeDtypeStruct((tokens, 1), x.dtype),
        ],
        compiler_params=pltpu.CompilerParams(
            dimension_semantics=("parallel",),
        ),
    )(x, w_all)
    return (s, g)


# speedup vs baseline: 1.1771x; 1.1771x over previous
import jax
import jax.numpy as jnp
from jax import lax
from jax.experimental import pallas as pl
from jax.experimental.pallas import tpu as pltpu

_D_MODEL = 4096
_NUM_EXPERTS = 64
_BLOCK_T = 1024


def _router_kernel(x_ref, w_ref, s_ref, g_ref):
    h = _BLOCK_T // 8
    w = w_ref[...]
    for p in range(8):
        sl = pl.ds(p * h, h)
        logits_all = lax.dot_general(
            x_ref[sl, :], w, (((1,), (1,)), ((), ())),
            preferred_element_type=jnp.float32)
        logits = logits_all[:, :_NUM_EXPERTS]
        m = jnp.max(logits, axis=-1, keepdims=True)
        e = jnp.exp(logits - m)
        s_ref[sl, :] = e / jnp.sum(e, axis=-1, keepdims=True)
        g_ref[sl, :] = jax.nn.sigmoid(logits_all[:, _NUM_EXPERTS:_NUM_EXPERTS + 1])


def kernel(x, W_router, W_shared_gate):
    tokens, d = x.shape
    n_exp = W_router.shape[0]
    w_all = jnp.concatenate(
        [W_router, W_shared_gate,
         jnp.zeros((128 - n_exp - 1, d), dtype=x.dtype)], axis=0)

    grid = (tokens // _BLOCK_T,)
    s, g = pl.pallas_call(
        _router_kernel,
        grid=grid,
        in_specs=[
            pl.BlockSpec((_BLOCK_T, d), lambda i: (i, 0)),
            pl.BlockSpec((128, d), lambda i: (0, 0)),
        ],
        out_specs=[
            pl.BlockSpec((_BLOCK_T, n_exp), lambda i: (i, 0)),
            pl.BlockSpec((_BLOCK_T, 1), lambda i: (i, 0)),
        ],
        out_shape=[
            jax.ShapeDtypeStruct((tokens, n_exp), x.dtype),
            jax.ShapeDtypeStruct((tokens, 1), x.dtype),
        ],
        compiler_params=pltpu.CompilerParams(
            dimension_semantics=("parallel",),
        ),
    )(x, w_all)
    return (s, g)
